# Initial kernel scaffold; baseline (speedup 1.0000x reference)
import jax
import jax.numpy as jnp
from jax.experimental import pallas as pl


def kernel(x, target):
    raise NotImplementedError("write your pallas kernel here")


# speedup vs baseline: 11.5768x; 11.5768x over previous
import functools

import jax
import jax.numpy as jnp
from jax.experimental import pallas as pl

_SIZE = 16384
_SMOOTH = 0.1
_CONF = 0.9
_S = _SMOOTH / (_SIZE - 2)


def _ls_kernel(t_ref, x_ref, o_ref, *, n_blocks):
    i = pl.program_id(0)
    xb = x_ref[...]
    tcol = t_ref[0]
    br, c = xb.shape
    colid = jax.lax.broadcasted_iota(jnp.int32, (br, c), 1)
    sel = colid == tcol
    xt = jnp.sum(jnp.where(sel, xb, 0.0), axis=1, keepdims=True)
    rowsum = jnp.sum(xb, axis=1, keepdims=True)
    x0 = xb[:, 0:1]
    k_const = _SMOOTH * jnp.log(_S) + _CONF * jnp.log(_CONF)
    contrib = jnp.where(
        tcol != 0,
        k_const - _S * rowsum + _S * x0 + (_S - _CONF) * xt,
        0.0,
    )
    total = jnp.sum(contrib).reshape(1, 1)

    @pl.when(i == 0)
    def _():
        o_ref[...] = jnp.zeros_like(o_ref)

    o_ref[...] += total


def kernel(x, target):
    n, c = x.shape
    br = 256
    n_blocks = n // br
    tr = target.reshape(n_blocks, br, 1)
    out = pl.pallas_call(
        functools.partial(_ls_kernel, n_blocks=n_blocks),
        grid=(n_blocks,),
        in_specs=[
            pl.BlockSpec((1, br, 1), lambda i: (i, 0, 0)),
            pl.BlockSpec((br, c), lambda i: (i, 0)),
        ],
        out_specs=pl.BlockSpec((1, 1), lambda i: (0, 0)),
        out_shape=jax.ShapeDtypeStruct((1, 1), jnp.float32),
    )(tr, x)
    return out[0, 0]
